# Initial kernel scaffold; baseline (speedup 1.0000x reference)
import jax
import jax.numpy as jnp
from jax.experimental import pallas as pl


def kernel(x, edge_index, W_l1, W_r1, b1, W_l2, W_r2, b2, W_ih3, W_hh3, b_ih3, b_hh3, W_l3, W_r3, b3, Wf_ih, Wf_hh, bf_ih, bf_hh, Wb_ih, Wb_hh, bb_ih, bb_hh, W_att, b_att):
    raise NotImplementedError("write your pallas kernel here")


# speedup vs baseline: 7.8994x; 7.8994x over previous
import functools

import jax
import jax.numpy as jnp
from jax import lax
from jax.experimental import pallas as pl
from jax.experimental.pallas import tpu as pltpu
from jax.experimental.pallas import tpu_sc as plsc

_NC = 2
_NS = 16
_NW = _NC * _NS
_CHK = 40
_RPW = 8
C = 8


def _sc_gather_chunk(table, src_s, idx3, *, E):
    NP = table.shape[0]
    D = table.shape[1]
    NROW = NP // _CHK
    RPW = _RPW

    mesh = plsc.VectorSubcoreMesh(core_axis_name="c", subcore_axis_name="s")

    @functools.partial(
        pl.kernel,
        mesh=mesh,
        out_type=jax.ShapeDtypeStruct((C, NROW, _CHK, D), jnp.float32),
        scratch_types=[
            pltpu.VMEM((RPW, _CHK), jnp.int32),
            pltpu.VMEM((RPW, _CHK), jnp.int32),
            pltpu.VMEM((RPW, _CHK, D), jnp.float32),
            pltpu.SemaphoreType.DMA,
        ],
    )
    def k(table_hbm, srcs_hbm, idx_hbm, out_hbm, idx_v, sid_v, rows_v, sem):
        wid = lax.axis_index("s") * _NC + lax.axis_index("c")
        row0 = wid * RPW
        for c in range(C):
            pltpu.sync_copy(idx_hbm.at[c, pl.ds(row0, RPW)], idx_v)
            h1 = [pltpu.async_copy(srcs_hbm.at[idx_v.at[j]], sid_v.at[j], sem)
                  for j in range(RPW)]
            for h in h1:
                h.wait()
            h2 = [pltpu.async_copy(table_hbm.at[sid_v.at[j]], rows_v.at[j], sem)
                  for j in range(RPW)]
            for h in h2:
                h.wait()
            pltpu.sync_copy(rows_v, out_hbm.at[c, pl.ds(row0, RPW)])

    return k(table, src_s, idx3)


def _tc_reduce(Xc, cnt_rel, acc, *, mode):
    NP, D = acc.shape
    blk = 1024

    def body(xc_ref, cnt_ref, acc_ref, out_ref):
        a = acc_ref[...]
        cnt = cnt_ref[...]
        for s in range(C):
            x = xc_ref[s]
            valid = cnt > s
            if mode == "sum":
                a = a + jnp.where(valid, x, 0.0)
            else:
                a = jnp.maximum(a, jnp.where(valid, x, -jnp.inf))
        out_ref[...] = a

    return pl.pallas_call(
        body,
        grid=(NP // blk,),
        in_specs=[
            pl.BlockSpec((C, blk, D), lambda i: (0, i, 0)),
            pl.BlockSpec((blk, 1), lambda i: (i, 0)),
            pl.BlockSpec((blk, D), lambda i: (i, 0)),
        ],
        out_specs=pl.BlockSpec((blk, D), lambda i: (i, 0)),
        out_shape=jax.ShapeDtypeStruct((NP, D), jnp.float32),
        input_output_aliases={2: 0},
    )(Xc, cnt_rel, acc)


def _tc_lstm_chunk(md_rel, Xc, cnt_rel, h, c, W_ih, W_hh, bias):
    NP, D = h.shape
    blk = 512

    def body(md_ref, xc_ref, cnt_ref, h_ref, c_ref, wi_ref, wh_ref, b_ref,
             ho_ref, co_ref):
        hh = h_ref[...]
        cc = c_ref[...]
        cnt = cnt_ref[...]
        wi = wi_ref[...]
        wh = wh_ref[...]
        b = b_ref[...]
        md = md_ref[0]
        for s in range(C):
            x = jnp.where(cnt > s, xc_ref[s], 0.0)
            g = (jnp.dot(x, wi, preferred_element_type=jnp.float32)
                 + jnp.dot(hh, wh, preferred_element_type=jnp.float32) + b)
            gi = jax.nn.sigmoid(g[:, 0 * D:1 * D])
            gf = jax.nn.sigmoid(g[:, 1 * D:2 * D])
            gg = jnp.tanh(g[:, 2 * D:3 * D])
            go = jax.nn.sigmoid(g[:, 3 * D:4 * D])
            cn = gf * cc + gi * gg
            hn = go * jnp.tanh(cn)
            upd = s < md
            hh = jnp.where(upd, hn, hh)
            cc = jnp.where(upd, cn, cc)
        ho_ref[...] = hh
        co_ref[...] = cc

    return pl.pallas_call(
        body,
        grid=(NP // blk,),
        in_specs=[
            pl.BlockSpec(memory_space=pltpu.SMEM),
            pl.BlockSpec((C, blk, D), lambda i: (0, i, 0)),
            pl.BlockSpec((blk, 1), lambda i: (i, 0)),
            pl.BlockSpec((blk, D), lambda i: (i, 0)),
            pl.BlockSpec((blk, D), lambda i: (i, 0)),
            pl.BlockSpec((D, 4 * D), lambda i: (0, 0)),
            pl.BlockSpec((D, 4 * D), lambda i: (0, 0)),
            pl.BlockSpec((1, 4 * D), lambda i: (0, 0)),
        ],
        out_specs=[
            pl.BlockSpec((blk, D), lambda i: (i, 0)),
            pl.BlockSpec((blk, D), lambda i: (i, 0)),
        ],
        out_shape=[
            jax.ShapeDtypeStruct((NP, D), jnp.float32),
            jax.ShapeDtypeStruct((NP, D), jnp.float32),
        ],
        input_output_aliases={3: 0, 4: 1},
    )(md_rel, Xc, cnt_rel, h, c, W_ih, W_hh, bias)


def _tc_combine(agg, hprev, Wl, Wr, b, cnt, *, mode):
    NP, D = agg.shape
    blk = 512

    def body(agg_ref, hp_ref, wl_ref, wr_ref, b_ref, cnt_ref, out_ref):
        a = agg_ref[...]
        cntf = cnt_ref[...].astype(jnp.float32)
        if mode == "mean":
            a = a / jnp.maximum(cntf, 1.0)
        elif mode == "max":
            a = jnp.where(cntf > 0.0, a, 0.0)
        o = (jnp.dot(a, wl_ref[...], preferred_element_type=jnp.float32)
             + jnp.dot(hp_ref[...], wr_ref[...],
                       preferred_element_type=jnp.float32)
             + b_ref[...])
        if mode != "plain":
            o = jnp.maximum(o, 0.0)
        out_ref[...] = o

    return pl.pallas_call(
        body,
        grid=(NP // blk,),
        in_specs=[
            pl.BlockSpec((blk, D), lambda i: (i, 0)),
            pl.BlockSpec((blk, D), lambda i: (i, 0)),
            pl.BlockSpec((D, D), lambda i: (0, 0)),
            pl.BlockSpec((D, D), lambda i: (0, 0)),
            pl.BlockSpec((1, D), lambda i: (0, 0)),
            pl.BlockSpec((blk, 1), lambda i: (i, 0)),
        ],
        out_specs=pl.BlockSpec((blk, D), lambda i: (i, 0)),
        out_shape=jax.ShapeDtypeStruct((NP, D), jnp.float32),
    )(agg, hprev, Wl, Wr, b, cnt)


def _tc_jk(h1, h2, h3, Wf_ih, Wf_hh, bf, Wb_ih, Wb_hh, bb, watt, *, H):
    NP, D = h1.shape
    blk = 512

    def body(h1_ref, h2_ref, h3_ref, wfi_ref, wfh_ref, bf_ref,
             wbi_ref, wbh_ref, bb_ref, wa_ref, out_ref):
        x1 = h1_ref[...]
        x2 = h2_ref[...]
        x3 = h3_ref[...]
        seq = (x1, x2, x3)

        def cell(x, h, c, wi, wh, b):
            g = (jnp.dot(x, wi, preferred_element_type=jnp.float32)
                 + jnp.dot(h, wh, preferred_element_type=jnp.float32) + b)
            gi = jax.nn.sigmoid(g[:, 0 * H:1 * H])
            gf = jax.nn.sigmoid(g[:, 1 * H:2 * H])
            gg = jnp.tanh(g[:, 2 * H:3 * H])
            go = jax.nn.sigmoid(g[:, 3 * H:4 * H])
            c2 = gf * c + gi * gg
            return go * jnp.tanh(c2), c2

        wfi = wfi_ref[...]
        wfh = wfh_ref[...]
        bfv = bf_ref[...]
        wbi = wbi_ref[...]
        wbh = wbh_ref[...]
        bbv = bb_ref[...]
        z = jnp.zeros((x1.shape[0], H), jnp.float32)
        hf, cf = z, z
        hs_f = []
        for t in range(3):
            hf, cf = cell(seq[t], hf, cf, wfi, wfh, bfv)
            hs_f.append(hf)
        hb, cb = z, z
        hs_b = [None, None, None]
        for k in range(3):
            t = 2 - k
            hb, cb = cell(seq[t], hb, cb, wbi, wbh, bbv)
            hs_b[t] = hb
        wa = wa_ref[...]
        wa_f = wa[:, :H]
        wa_b = wa[:, H:]
        atts = []
        for t in range(3):
            att = (jnp.sum(hs_f[t] * wa_f, axis=1, keepdims=True)
                   + jnp.sum(hs_b[t] * wa_b, axis=1, keepdims=True))
            atts.append(att)
        m = jnp.maximum(atts[0], jnp.maximum(atts[1], atts[2]))
        e0 = jnp.exp(atts[0] - m)
        e1 = jnp.exp(atts[1] - m)
        e2 = jnp.exp(atts[2] - m)
        z_sum = e0 + e1 + e2
        out_ref[...] = (e0 * x1 + e1 * x2 + e2 * x3) / z_sum

    return pl.pallas_call(
        body,
        grid=(NP // blk,),
        in_specs=[
            pl.BlockSpec((blk, D), lambda i: (i, 0)),
            pl.BlockSpec((blk, D), lambda i: (i, 0)),
            pl.BlockSpec((blk, D), lambda i: (i, 0)),
            pl.BlockSpec((D, 4 * H), lambda i: (0, 0)),
            pl.BlockSpec((H, 4 * H), lambda i: (0, 0)),
            pl.BlockSpec((1, 4 * H), lambda i: (0, 0)),
            pl.BlockSpec((D, 4 * H), lambda i: (0, 0)),
            pl.BlockSpec((H, 4 * H), lambda i: (0, 0)),
            pl.BlockSpec((1, 4 * H), lambda i: (0, 0)),
            pl.BlockSpec((1, 2 * H), lambda i: (0, 0)),
        ],
        out_specs=pl.BlockSpec((blk, D), lambda i: (i, 0)),
        out_shape=jax.ShapeDtypeStruct((NP, D), jnp.float32),
    )(h1, h2, h3, Wf_ih, Wf_hh, bf, Wb_ih, Wb_hh, bb, watt)


def _make_idx(starts_p, t0, *, E, NP):
    offs = t0 + jnp.arange(C, dtype=jnp.int32)
    idx = jnp.minimum(starts_p[None, :] + offs[:, None], E - 1)
    return idx.reshape(C, NP // _CHK, _CHK)


def _agg_pass(table, src_s, starts_p, counts_p, max_deg, *, mode, E):
    NP, D = table.shape
    if mode == "sum":
        init = jnp.zeros((NP, D), jnp.float32)
    else:
        init = jnp.full((NP, D), -jnp.inf, jnp.float32)
    K = (max_deg + C - 1) // C

    def cond(st):
        return st[0] < K

    def body(st):
        i, acc = st
        t0 = i * C
        idx3 = _make_idx(starts_p, t0, E=E, NP=NP)
        Xc = _sc_gather_chunk(table, src_s, idx3, E=E).reshape(C, NP, D)
        cnt_rel = (counts_p - t0)[:, None]
        acc = _tc_reduce(Xc, cnt_rel, acc, mode=mode)
        return (i + jnp.int32(1), acc)

    _, acc = lax.while_loop(cond, body, (jnp.int32(0), init))
    return acc


def _lstm_pass(table, src_s, starts_p, counts_p, max_deg, W_ih, W_hh, bias,
               *, E):
    NP, D = table.shape
    K = (max_deg + C - 1) // C
    h0 = jnp.zeros((NP, D), jnp.float32)
    c0 = jnp.zeros((NP, D), jnp.float32)

    def cond(st):
        return st[0] < K

    def body(st):
        i, h, c = st
        t0 = i * C
        idx3 = _make_idx(starts_p, t0, E=E, NP=NP)
        Xc = _sc_gather_chunk(table, src_s, idx3, E=E).reshape(C, NP, D)
        cnt_rel = (counts_p - t0)[:, None]
        md_rel = jnp.reshape(max_deg - t0, (1,)).astype(jnp.int32)
        h, c = _tc_lstm_chunk(md_rel, Xc, cnt_rel, h, c, W_ih, W_hh, bias)
        return (i + jnp.int32(1), h, c)

    _, h, _ = lax.while_loop(cond, body, (jnp.int32(0), h0, c0))
    return h


def kernel(x, edge_index, W_l1, W_r1, b1, W_l2, W_r2, b2, W_ih3, W_hh3, b_ih3,
           b_hh3, W_l3, W_r3, b3, Wf_ih, Wf_hh, bf_ih, bf_hh, Wb_ih, Wb_hh,
           bb_ih, bb_hh, W_att, b_att):
    N, D = x.shape
    E = edge_index.shape[1]
    H = Wf_hh.shape[0]
    GR = _NW * _RPW * _CHK
    NP = ((N + GR - 1) // GR) * GR

    src = edge_index[0].astype(jnp.int32)
    dst = edge_index[1].astype(jnp.int32)
    perm = jnp.argsort(dst, stable=True)
    src_s = src[perm]
    counts = jnp.bincount(dst, length=N).astype(jnp.int32)
    starts = jnp.concatenate(
        [jnp.zeros((1,), jnp.int32), jnp.cumsum(counts)[:-1].astype(jnp.int32)])
    max_deg = jnp.max(counts)

    x_p = jnp.zeros((NP, D), jnp.float32).at[:N].set(x)
    counts_p = jnp.zeros((NP,), jnp.int32).at[:N].set(counts)
    starts_p = jnp.zeros((NP,), jnp.int32).at[:N].set(starts)
    cnt_col = counts_p[:, None]

    b1r = b1.reshape(1, D)
    b2r = b2.reshape(1, D)
    b3r = b3.reshape(1, D)
    b3s = (b_ih3 + b_hh3).reshape(1, 4 * D)
    bfr = (bf_ih + bf_hh).reshape(1, 4 * H)
    bbr = (bb_ih + bb_hh).reshape(1, 4 * H)
    watt = W_att.reshape(1, 2 * H)

    agg1 = _agg_pass(x_p, src_s, starts_p, counts_p, max_deg, mode="sum", E=E)
    h1 = _tc_combine(agg1, x_p, W_l1, W_r1, b1r, cnt_col, mode="mean")
    agg2 = _agg_pass(h1, src_s, starts_p, counts_p, max_deg, mode="max", E=E)
    h2 = _tc_combine(agg2, h1, W_l2, W_r2, b2r, cnt_col, mode="max")
    m3 = _lstm_pass(h2, src_s, starts_p, counts_p, max_deg, W_ih3, W_hh3, b3s,
                    E=E)
    h3 = _tc_combine(m3, h2, W_l3, W_r3, b3r, cnt_col, mode="plain")
    hout = _tc_jk(h1, h2, h3, Wf_ih, Wf_hh, bfr, Wb_ih, Wb_hh, bbr, watt, H=H)
    h = hout[:N]
    return (h, h)
